# Initial kernel scaffold; baseline (speedup 1.0000x reference)
import jax
import jax.numpy as jnp
from jax.experimental import pallas as pl


def kernel(x, y):
    raise NotImplementedError("write your pallas kernel here")


# speedup vs baseline: 23.3450x; 23.3450x over previous
import functools

import jax
import jax.numpy as jnp
import numpy as np
from jax.experimental import pallas as pl

_HARD_P = 0.5
_RAND_P = 0.1

_RAND_CACHE = {}


def _random_mask_const(b, hw):
    key = (b, hw)
    if key not in _RAND_CACHE:
        n_ones = int(_RAND_P * hw)
        with jax.ensure_compile_time_eval():
            base = jnp.concatenate([
                jnp.ones((n_ones,), dtype=jnp.float32),
                jnp.zeros((hw - n_ones,), dtype=jnp.float32),
            ])
            keys = jax.random.split(jax.random.key(42), b)
            rm = jax.vmap(lambda k: jax.random.permutation(k, base))(keys)
        _RAND_CACHE[key] = np.asarray(rm)
    return _RAND_CACHE[key]


def _res_body(x_ref, y_ref, out_ref):
    out_ref[...] = jnp.sum(jnp.abs(x_ref[...] - y_ref[...]), axis=1)


def _residual(x, y):
    b, c, h, w = x.shape
    rows = 32
    grid = (b, h // rows)
    return pl.pallas_call(
        _res_body,
        grid=grid,
        in_specs=[
            pl.BlockSpec((1, c, rows, w), lambda i, j: (i, 0, j, 0)),
            pl.BlockSpec((1, c, rows, w), lambda i, j: (i, 0, j, 0)),
        ],
        out_specs=pl.BlockSpec((1, rows, w), lambda i, j: (i, j, 0)),
        out_shape=jax.ShapeDtypeStruct((b, h, w), jnp.float32),
    )(x, y)


def _select_body(res_ref, rand_ref, out_ref, *, kth, inv_n, nb):
    bidx = pl.program_id(0)
    u = jax.lax.bitcast_convert_type(res_ref[0], jnp.int32)

    def step(i, prefix):
        cand = prefix | (jnp.int32(1) << (jnp.int32(30) - i))
        cnt = jnp.sum((u >= cand).astype(jnp.int32))
        return jnp.where(cnt >= kth + 1, cand, prefix)

    thre_bits = jax.lax.fori_loop(0, 31, step, jnp.int32(0))
    keep = (u > thre_bits) | (rand_ref[0] > 0.0)
    contrib = jnp.sum(jnp.where(keep, res_ref[0], 0.0)) * inv_n

    @pl.when(bidx == 0)
    def _():
        out_ref[...] = jnp.zeros_like(out_ref)

    out_ref[...] += jnp.full((1, 1), contrib, jnp.float32)


def _select_and_sum(res2d, rand2d, kth, n_total):
    b, nrow, ncol = res2d.shape
    body = functools.partial(
        _select_body, kth=kth, inv_n=np.float32(1.0 / n_total), nb=b)
    return pl.pallas_call(
        body,
        grid=(b,),
        in_specs=[
            pl.BlockSpec((1, nrow, ncol), lambda i: (i, 0, 0)),
            pl.BlockSpec((1, nrow, ncol), lambda i: (i, 0, 0)),
        ],
        out_specs=pl.BlockSpec((1, 1), lambda i: (0, 0)),
        out_shape=jax.ShapeDtypeStruct((1, 1), jnp.float32),
    )(res2d, rand2d)


def kernel(x, y):
    b, c, h, w = x.shape
    hw = h * w
    kth = int(_HARD_P * hw)
    res = _residual(x, y)
    ncol = 128
    nrow = hw // ncol
    res2d = res.reshape(b, nrow, ncol)
    rand2d = jnp.asarray(_random_mask_const(b, hw).reshape(b, nrow, ncol))
    out = _select_and_sum(res2d, rand2d, kth, b * c * hw)
    return out[0, 0]
